# baseline (speedup 1.0000x reference)
import functools

import jax
import jax.numpy as jnp
from jax import lax
from jax.experimental import pallas as pl
from jax.experimental.pallas import tpu as pltpu
from jax.experimental.pallas import tpu_sc as plsc

_NUM_TOKENS = 8192
_CODE_DIM = 256
_BETA = 0.25
_TN = 256
_GRID = _NUM_TOKENS // _TN
_IDX_CHUNK = 128


def _vq_body(z_ref, w_ref, wsq_ref, enc_ref, idx_ref, loss_ref, perp_ref,
             counts_ref):
    i = pl.program_id(0)
    zb = z_ref[...].reshape(_CODE_DIM, _TN)
    zf = zb.T
    w = w_ref[...]

    scores = jax.lax.dot_general(
        zf, w, (((1,), (1,)), ((), ())),
        preferred_element_type=jnp.float32)
    zsq = jnp.sum(zf * zf, axis=1, keepdims=True)
    d = (zsq + wsq_ref[...]) - 2.0 * scores

    m = jnp.min(d, axis=1, keepdims=True)
    col = jax.lax.broadcasted_iota(jnp.int32, (_TN, _NUM_TOKENS), 1)
    idx = jnp.min(jnp.where(d == m, col, jnp.int32(2**30)), axis=1)
    idx_ref[0, 0, :] = idx

    enc = (col == idx[:, None]).astype(jnp.float32)
    enc_ref[...] = enc

    ones_row = jnp.ones((1, _TN), jnp.float32)
    part_counts = jax.lax.dot_general(
        ones_row, enc, (((1,), (0,)), ((), ())),
        preferred_element_type=jnp.float32)

    part_loss = jnp.sum(m).reshape(1, 1)

    @pl.when(i == 0)
    def _init():
        counts_ref[...] = part_counts
        loss_ref[...] = part_loss

    @pl.when(i > 0)
    def _acc():
        counts_ref[...] = counts_ref[...] + part_counts
        loss_ref[...] = loss_ref[...] + part_loss

    @pl.when(i == _GRID - 1)
    def _finalize():
        loss_ref[...] = loss_ref[...] * (_BETA / (_NUM_TOKENS * _CODE_DIM))
        avg = counts_ref[...] * (1.0 / _NUM_TOKENS)
        perp_ref[...] = jnp.exp(
            -jnp.sum(avg * jnp.log(avg + 1e-10))).reshape(1, 1)


def _sc_gather_body(table_hbm, idx_hbm, out_hbm, idx_v, rows_v, sem):
    nc = 2
    wid = lax.axis_index("s") * nc + lax.axis_index("c")
    b_per_w = _NUM_TOKENS // 32
    base = wid * b_per_w
    for j in range(b_per_w // _IDX_CHUNK):
        off = base + j * _IDX_CHUNK
        pltpu.sync_copy(idx_hbm.at[pl.ds(off, _IDX_CHUNK)], idx_v)
        pltpu.async_copy(table_hbm.at[idx_v], rows_v, sem).wait()
        pltpu.sync_copy(rows_v, out_hbm.at[pl.ds(off, _IDX_CHUNK)])


def _sc_gather(W, idx):
    mesh = plsc.VectorSubcoreMesh(core_axis_name="c", subcore_axis_name="s")
    f = functools.partial(
        pl.kernel, mesh=mesh,
        out_type=jax.ShapeDtypeStruct((_NUM_TOKENS, _CODE_DIM), jnp.float32),
        scratch_types=[
            pltpu.VMEM((_IDX_CHUNK,), jnp.int32),
            pltpu.VMEM((_IDX_CHUNK, _CODE_DIM), jnp.float32),
            pltpu.SemaphoreType.DMA,
        ],
    )(_sc_gather_body)
    return f(W, idx)


def kernel(z, W):
    b, c, h, w = z.shape
    hrows = _TN // w
    wsq = jnp.sum(W * W, axis=1)[None, :]

    enc, idx3, loss, perp = pl.pallas_call(
        _vq_body,
        grid=(_GRID,),
        in_specs=[
            pl.BlockSpec((1, _CODE_DIM, hrows, w),
                         lambda i: (i // 4, 0, i % 4, 0)),
            pl.BlockSpec((_NUM_TOKENS, _CODE_DIM), lambda i: (0, 0)),
            pl.BlockSpec((1, _NUM_TOKENS), lambda i: (0, 0)),
        ],
        out_specs=[
            pl.BlockSpec((_TN, _NUM_TOKENS), lambda i: (i, 0)),
            pl.BlockSpec((1, 1, _TN), lambda i: (i, 0, 0)),
            pl.BlockSpec((1, 1), lambda i: (0, 0)),
            pl.BlockSpec((1, 1), lambda i: (0, 0)),
        ],
        out_shape=[
            jax.ShapeDtypeStruct((_NUM_TOKENS, _NUM_TOKENS), jnp.float32),
            jax.ShapeDtypeStruct((_GRID, 1, _TN), jnp.int32),
            jax.ShapeDtypeStruct((1, 1), jnp.float32),
            jax.ShapeDtypeStruct((1, 1), jnp.float32),
        ],
        scratch_shapes=[pltpu.VMEM((1, _NUM_TOKENS), jnp.float32)],
    )(z, W, wsq)

    encoding_indices = idx3.reshape(-1)
    zq = _sc_gather(W, encoding_indices)
    z_q_out = jnp.transpose(zq.reshape(b, h, w, c), (0, 3, 1, 2))
    return (z_q_out, loss[0, 0], perp[0, 0], enc, encoding_indices)


# speedup vs baseline: 1.1968x; 1.1968x over previous
import functools

import jax
import jax.numpy as jnp
from jax import lax
from jax.experimental import pallas as pl
from jax.experimental.pallas import tpu as pltpu
from jax.experimental.pallas import tpu_sc as plsc

_NUM_TOKENS = 8192
_CODE_DIM = 256
_BETA = 0.25
_TN = 256
_GRID = _NUM_TOKENS // _TN
_IDX_CHUNK = 128


def _vq_body(zf_ref, w_ref, wsq_ref, enc_ref, idx_ref, loss_ref, perp_ref,
             counts_ref):
    i = pl.program_id(0)
    zf = zf_ref[...]
    w = w_ref[...]

    scores = jax.lax.dot_general(
        zf, w, (((1,), (1,)), ((), ())),
        preferred_element_type=jnp.float32)
    zsq = jnp.sum(zf * zf, axis=1, keepdims=True)
    d = (zsq + wsq_ref[...]) - 2.0 * scores

    m = jnp.min(d, axis=1, keepdims=True)
    col = jax.lax.broadcasted_iota(jnp.int32, (_TN, _NUM_TOKENS), 1)
    idx = jnp.min(jnp.where(d == m, col, jnp.int32(2**30)), axis=1)
    idx_ref[0, 0, :] = idx

    enc = (col == idx[:, None]).astype(jnp.float32)
    enc_ref[...] = enc

    ones_row = jnp.ones((1, _TN), jnp.float32)
    part_counts = jax.lax.dot_general(
        ones_row, enc, (((1,), (0,)), ((), ())),
        preferred_element_type=jnp.float32)

    part_loss = jnp.sum(m).reshape(1, 1)

    @pl.when(i == 0)
    def _init():
        counts_ref[...] = part_counts
        loss_ref[...] = part_loss

    @pl.when(i > 0)
    def _acc():
        counts_ref[...] = counts_ref[...] + part_counts
        loss_ref[...] = loss_ref[...] + part_loss

    @pl.when(i == _GRID - 1)
    def _finalize():
        loss_ref[...] = loss_ref[...] * (_BETA / (_NUM_TOKENS * _CODE_DIM))
        avg = counts_ref[...] * (1.0 / _NUM_TOKENS)
        perp_ref[...] = jnp.exp(
            -jnp.sum(avg * jnp.log(avg + 1e-10))).reshape(1, 1)


def _sc_gather_body(table_hbm, idx_hbm, out_hbm, idx_v, rows_v, sem):
    nc = 2
    wid = lax.axis_index("s") * nc + lax.axis_index("c")
    b_per_w = _NUM_TOKENS // 32
    base = wid * b_per_w
    for j in range(b_per_w // _IDX_CHUNK):
        off = base + j * _IDX_CHUNK
        pltpu.sync_copy(idx_hbm.at[pl.ds(off, _IDX_CHUNK)], idx_v)
        pltpu.async_copy(table_hbm.at[idx_v], rows_v, sem).wait()
        pltpu.sync_copy(rows_v, out_hbm.at[pl.ds(off, _IDX_CHUNK)])


def _sc_gather(W, idx):
    mesh = plsc.VectorSubcoreMesh(core_axis_name="c", subcore_axis_name="s")
    f = functools.partial(
        pl.kernel, mesh=mesh,
        out_type=jax.ShapeDtypeStruct((_NUM_TOKENS, _CODE_DIM), jnp.float32),
        scratch_types=[
            pltpu.VMEM((_IDX_CHUNK,), jnp.int32),
            pltpu.VMEM((_IDX_CHUNK, _CODE_DIM), jnp.float32),
            pltpu.SemaphoreType.DMA,
        ],
    )(_sc_gather_body)
    return f(W, idx)


def kernel(z, W):
    b, c, h, w = z.shape
    zp = jnp.transpose(z, (0, 2, 3, 1))
    zf = zp.reshape(-1, _CODE_DIM)
    wsq = jnp.sum(W * W, axis=1)[None, :]

    enc, idx3, loss, perp = pl.pallas_call(
        _vq_body,
        grid=(_GRID,),
        in_specs=[
            pl.BlockSpec((_TN, _CODE_DIM), lambda i: (i, 0)),
            pl.BlockSpec((_NUM_TOKENS, _CODE_DIM), lambda i: (0, 0)),
            pl.BlockSpec((1, _NUM_TOKENS), lambda i: (0, 0)),
        ],
        out_specs=[
            pl.BlockSpec((_TN, _NUM_TOKENS), lambda i: (i, 0)),
            pl.BlockSpec((1, 1, _TN), lambda i: (i, 0, 0)),
            pl.BlockSpec((1, 1), lambda i: (0, 0)),
            pl.BlockSpec((1, 1), lambda i: (0, 0)),
        ],
        out_shape=[
            jax.ShapeDtypeStruct((_NUM_TOKENS, _NUM_TOKENS), jnp.float32),
            jax.ShapeDtypeStruct((_GRID, 1, _TN), jnp.int32),
            jax.ShapeDtypeStruct((1, 1), jnp.float32),
            jax.ShapeDtypeStruct((1, 1), jnp.float32),
        ],
        scratch_shapes=[pltpu.VMEM((1, _NUM_TOKENS), jnp.float32)],
    )(zf, W, wsq)

    encoding_indices = idx3.reshape(-1)
    zq = _sc_gather(W, encoding_indices)
    z_q_out = jnp.transpose(zq.reshape(b, h, w, c), (0, 3, 1, 2))
    return (z_q_out, loss[0, 0], perp[0, 0], enc, encoding_indices)
